# baseline (speedup 1.0000x reference)
import functools

import jax
import jax.numpy as jnp
from jax import lax
from jax.experimental import pallas as pl
from jax.experimental.pallas import tpu as pltpu
from jax.experimental.pallas import tpu_sc as plsc

VOCAB = 1000000
D = 64
N_TOK = 10
B = 4096
S = 200
G = S - N_TOK
GPAD = 192
C0 = 96
BUF_S = N_TOK + GPAD

NC = 2
NS = 16
NW = NC * NS
BPW = B // NW
NB = 4
NBLK = BPW // NB

_mesh = plsc.VectorSubcoreMesh(core_axis_name="c", subcore_axis_name="s")


@functools.partial(
    pl.kernel,
    mesh=_mesh,
    out_type=jax.ShapeDtypeStruct((B, S, D), jnp.float32),
    scratch_types=[
        pltpu.VMEM((NB, GPAD), jnp.int32),
        pltpu.VMEM((NB, GPAD), jnp.int32),
        pltpu.VMEM((NB, BUF_S, D), jnp.float32),
        pltpu.VMEM((NB, BUF_S, D), jnp.float32),
        pltpu.SemaphoreType.DMA,
        pltpu.SemaphoreType.DMA,
        pltpu.SemaphoreType.DMA,
        pltpu.SemaphoreType.DMA,
    ],
    compiler_params=pltpu.CompilerParams(use_tc_tiling_on_sc=False),
)
def _soft_embed(tok_hbm, wte_hbm, soft_hbm, out_hbm,
                idx_a, idx_b, buf_a, buf_b, sga, sgb, ssa, ssb):
    wid = lax.axis_index("s") * NC + lax.axis_index("c")
    base = wid * BPW

    for buf in (buf_a, buf_b):
        for i in range(NB):
            pltpu.sync_copy(soft_hbm, buf.at[i, pl.ds(0, N_TOK)])

    def fire_gathers(b0, idx, buf, sem):
        pltpu.sync_copy(tok_hbm.at[pl.ds(b0, NB)], idx)
        for i in range(NB):
            pltpu.async_copy(wte_hbm.at[idx.at[i, pl.ds(0, C0)]],
                             buf.at[i, pl.ds(N_TOK, C0)], sem)
            pltpu.async_copy(wte_hbm.at[idx.at[i, pl.ds(C0, C0)]],
                             buf.at[i, pl.ds(N_TOK + C0, C0)], sem)

    def wait_gathers(idx, buf, sem):
        for i in range(NB):
            pltpu.make_async_copy(wte_hbm.at[idx.at[i, pl.ds(0, C0)]],
                                  buf.at[i, pl.ds(N_TOK, C0)], sem).wait()
            pltpu.make_async_copy(wte_hbm.at[idx.at[i, pl.ds(C0, C0)]],
                                  buf.at[i, pl.ds(N_TOK + C0, C0)], sem).wait()

    def fire_stores(b0, buf, sem):
        for i in range(NB):
            pltpu.async_copy(buf.at[i, pl.ds(0, S)], out_hbm.at[b0 + i], sem)

    def wait_stores(b0, buf, sem):
        for i in range(NB):
            pltpu.make_async_copy(buf.at[i, pl.ds(0, S)],
                                  out_hbm.at[b0 + i], sem).wait()

    T = NBLK // 2

    fire_gathers(base, idx_a, buf_a, sga)

    def body(t, carry):
        a0 = base + (2 * t) * NB
        b0 = a0 + NB

        @pl.when(t > 0)
        def _():
            wait_stores(b0, buf_b, ssb)
        fire_gathers(b0, idx_b, buf_b, sgb)
        wait_gathers(idx_a, buf_a, sga)
        fire_stores(a0, buf_a, ssa)

        @pl.when(t + 1 < T)
        def _():
            wait_stores(a0, buf_a, ssa)
            fire_gathers(a0 + 2 * NB, idx_a, buf_a, sga)
        wait_gathers(idx_b, buf_b, sgb)
        fire_stores(b0, buf_b, ssb)
        return carry

    lax.fori_loop(0, T, body, 0)

    wait_stores(base, buf_a, ssa)
    wait_stores(base, buf_b, ssb)


def kernel(tokens, wte_weight, soft_embedding_weight):
    tok = tokens.astype(jnp.int32)[:, N_TOK:]
    tok = jnp.pad(tok, ((0, 0), (0, GPAD - G)))
    return _soft_embed(tok, wte_weight, soft_embedding_weight)
